# baseline (speedup 1.0000x reference)
import jax
import jax.numpy as jnp
from jax import lax
from jax.experimental import pallas as pl
from jax.experimental.pallas import tpu as pltpu
from jax.experimental.pallas import tpu_sc as plsc

VOCAB = 100000
HIDDEN = 768
MAX_POS = 2048
BATCH = 4
SEQ = 2048
EPS = 1e-12

NC = 2
NS = 16
L = 16
NW = NC * NS
CHUNK = SEQ // NW
NCH = HIDDEN // L


def _rsqrt(x):
    i = lax.bitcast_convert_type(x, jnp.int32)
    y = lax.bitcast_convert_type(0x5F3759DF - (i >> 1), jnp.float32)
    for _ in range(3):
        y = y * (1.5 - 0.5 * x * y * y)
    return y


_GATHER_DNUMS = lax.GatherDimensionNumbers(
    offset_dims=(), collapsed_slice_dims=(0,), start_index_map=(0,))


def _lane_shuffle(x, perm):
    return lax.gather(x, perm[:, None], _GATHER_DNUMS, slice_sizes=(1,),
                      mode=lax.GatherScatterMode.PROMISE_IN_BOUNDS)


def _allreduce_sum(x):
    lanes = lax.iota(jnp.int32, L)
    for k in range(4):
        x = x + _lane_shuffle(x, lanes ^ (1 << k))
    return x


def _body(ids_hbm, word_hbm, pos_hbm, lnw_hbm, lnb_hbm, out_hbm,
          idx_v, pos_v, rows_v, w_v, b_v, sem):
    wid = lax.axis_index("s") * NC + lax.axis_index("c")
    pbase = wid * CHUNK
    pltpu.sync_copy(pos_hbm.at[pl.ds(pbase, CHUNK)], pos_v)
    pltpu.sync_copy(lnw_hbm, w_v)
    pltpu.sync_copy(lnb_hbm, b_v)

    for b in range(BATCH):
        base = b * SEQ + pbase
        pltpu.sync_copy(ids_hbm.at[pl.ds(base, CHUNK)], idx_v)
        pltpu.async_copy(word_hbm.at[idx_v], rows_v, sem).wait()

        def row(r, carry):
            xs = []
            s = jnp.zeros((L,), jnp.float32)
            s2 = jnp.zeros((L,), jnp.float32)
            for i in range(NCH):
                x = rows_v[r, pl.ds(i * L, L)] + pos_v[r, pl.ds(i * L, L)]
                xs.append(x)
                s = s + x
                s2 = s2 + x * x
            u = _allreduce_sum(s) * (1.0 / HIDDEN)
            var = _allreduce_sum(s2) * (1.0 / HIDDEN) - u * u
            inv = _rsqrt(jnp.maximum(var, 0.0) + EPS)
            for i in range(NCH):
                y = (xs[i] - u) * inv
                y = y * w_v[pl.ds(i * L, L)] + b_v[pl.ds(i * L, L)]
                rows_v[r, pl.ds(i * L, L)] = y
            return carry

        lax.fori_loop(0, CHUNK, row, 0)
        pltpu.sync_copy(rows_v, out_hbm.at[pl.ds(base, CHUNK)])


def kernel(input_ids, attention_mask, word_emb, pos_emb, ln_w, ln_b):
    ids = input_ids.reshape(-1).astype(jnp.int32)
    mesh = plsc.VectorSubcoreMesh(core_axis_name="c", subcore_axis_name="s",
                                  num_cores=NC, num_subcores=NS)
    out = pl.kernel(
        _body,
        out_type=jax.ShapeDtypeStruct((BATCH * SEQ, HIDDEN), jnp.float32),
        mesh=mesh,
        scratch_types=[
            pltpu.VMEM((CHUNK,), jnp.int32),
            pltpu.VMEM((CHUNK, HIDDEN), jnp.float32),
            pltpu.VMEM((CHUNK, HIDDEN), jnp.float32),
            pltpu.VMEM((HIDDEN,), jnp.float32),
            pltpu.VMEM((HIDDEN,), jnp.float32),
            pltpu.SemaphoreType.DMA,
        ],
    )(ids, word_emb, pos_emb, ln_w, ln_b)
    return out.reshape(BATCH, SEQ, HIDDEN)


# speedup vs baseline: 1.0355x; 1.0355x over previous
import jax
import jax.numpy as jnp
from jax import lax
from jax.experimental import pallas as pl
from jax.experimental.pallas import tpu as pltpu
from jax.experimental.pallas import tpu_sc as plsc

VOCAB = 100000
HIDDEN = 768
MAX_POS = 2048
BATCH = 4
SEQ = 2048
EPS = 1e-12

NC = 2
NS = 16
L = 16
NW = NC * NS
CHUNK = SEQ // NW
NCH = HIDDEN // L
RPC = 32
NCHUNK = BATCH * CHUNK // RPC
NBUF = 3


def _rsqrt(x):
    i = lax.bitcast_convert_type(x, jnp.int32)
    y = lax.bitcast_convert_type(0x5F3759DF - (i >> 1), jnp.float32)
    for _ in range(3):
        y = y * (1.5 - 0.5 * x * y * y)
    return y


_GATHER_DNUMS = lax.GatherDimensionNumbers(
    offset_dims=(), collapsed_slice_dims=(0,), start_index_map=(0,))


def _lane_shuffle(x, perm):
    return lax.gather(x, perm[:, None], _GATHER_DNUMS, slice_sizes=(1,),
                      mode=lax.GatherScatterMode.PROMISE_IN_BOUNDS)


def _allreduce_sum(x):
    lanes = lax.iota(jnp.int32, L)
    for k in range(4):
        x = x + _lane_shuffle(x, lanes ^ (1 << k))
    return x


def _body(ids_hbm, word_hbm, pos_hbm, lnw_hbm, lnb_hbm, out_hbm,
          idx_v, pos_v, rows_v, w_v, b_v, gsems, wsems):
    wid = lax.axis_index("s") * NC + lax.axis_index("c")
    pbase = wid * CHUNK

    def hbase(c):
        return (c // 2) * SEQ + pbase + (c % 2) * RPC

    pltpu.sync_copy(lnw_hbm, w_v)
    pltpu.sync_copy(lnb_hbm, b_v)
    for c in range(NCHUNK):
        pltpu.sync_copy(ids_hbm.at[pl.ds(hbase(c), RPC)], idx_v.at[c])
    pltpu.sync_copy(pos_hbm.at[pl.ds(pbase, CHUNK)], pos_v)

    def start_gather(c):
        return pltpu.async_copy(word_hbm.at[idx_v.at[c]],
                                rows_v.at[c % NBUF], gsems.at[c % NBUF])

    gathers = {0: start_gather(0), 1: start_gather(1)}
    writes = {}

    for c in range(NCHUNK):
        buf = c % NBUF
        nxt = c + NBUF - 1
        if nxt < NCHUNK:
            if nxt - NBUF in writes:
                writes[nxt - NBUF].wait()
            gathers[nxt] = start_gather(nxt)
        gathers[c].wait()

        poff = (c % 2) * RPC
        buf_ref = rows_v.at[buf]

        @plsc.parallel_loop(0, RPC, unroll=2)
        def row(r):
            s = jnp.zeros((L,), jnp.float32)
            s2 = jnp.zeros((L,), jnp.float32)
            for i in range(NCH):
                x = buf_ref[r, pl.ds(i * L, L)] + pos_v[poff + r, pl.ds(i * L, L)]
                buf_ref[r, pl.ds(i * L, L)] = x
                s = s + x
                s2 = s2 + x * x
            u = _allreduce_sum(s) * (1.0 / HIDDEN)
            var = _allreduce_sum(s2) * (1.0 / HIDDEN) - u * u
            inv = _rsqrt(jnp.maximum(var, 0.0) + EPS)
            for i in range(NCH):
                x = buf_ref[r, pl.ds(i * L, L)]
                y = (x - u) * inv
                y = y * w_v[pl.ds(i * L, L)] + b_v[pl.ds(i * L, L)]
                buf_ref[r, pl.ds(i * L, L)] = y

        writes[c] = pltpu.async_copy(buf_ref, out_hbm.at[pl.ds(hbase(c), RPC)],
                                     wsems.at[buf])

    for c in range(NCHUNK - NBUF, NCHUNK):
        writes[c].wait()


def kernel(input_ids, attention_mask, word_emb, pos_emb, ln_w, ln_b):
    ids = input_ids.reshape(-1).astype(jnp.int32)
    mesh = plsc.VectorSubcoreMesh(core_axis_name="c", subcore_axis_name="s",
                                  num_cores=NC, num_subcores=NS)
    out = pl.kernel(
        _body,
        out_type=jax.ShapeDtypeStruct((BATCH * SEQ, HIDDEN), jnp.float32),
        mesh=mesh,
        scratch_types=[
            pltpu.VMEM((NCHUNK, RPC), jnp.int32),
            pltpu.VMEM((CHUNK, HIDDEN), jnp.float32),
            pltpu.VMEM((NBUF, RPC, HIDDEN), jnp.float32),
            pltpu.VMEM((HIDDEN,), jnp.float32),
            pltpu.VMEM((HIDDEN,), jnp.float32),
            pltpu.SemaphoreType.DMA((NBUF,)),
            pltpu.SemaphoreType.DMA((NBUF,)),
        ],
    )(ids, word_emb, pos_emb, ln_w, ln_b)
    return out.reshape(BATCH, SEQ, HIDDEN)
